# baseline (speedup 1.0000x reference)
import jax
import jax.numpy as jnp
import numpy as np
from jax.experimental import pallas as pl
from jax.experimental.pallas import tpu as pltpu

_N = 4096
_DIN = 256
_DOUT = 64
_H = 4
_ALPHA = 0.2
_EXT = 128
_LOG2E = float(np.log2(np.e))

_RB = 512


def _att_kernel(x_ref, adj_ref, wcat_ref, a1_ref, a2_ref, out_ref,
                whext_ref, e2_ref, e2s_ref, b1_ref, b2_ref):
    i = pl.program_id(0)

    @pl.when(i == 0)
    def _prologue():
        wh = jnp.dot(x_ref[...], wcat_ref[...],
                     preferred_element_type=jnp.float32)
        whb = wh.astype(jnp.bfloat16)
        ones = jnp.ones((_N, 1), jnp.bfloat16)
        zpad = jnp.zeros((_N, _EXT - _DOUT - 1), jnp.bfloat16)
        for h in range(_H):
            whext_ref[:, h * _EXT:(h + 1) * _EXT] = jnp.concatenate(
                [whb[:, h * _DOUT:(h + 1) * _DOUT], ones, zpad], axis=1)
        e1t = jax.lax.dot_general(
            a1_ref[...], wh, (((0,), (1,)), ((), ())),
            preferred_element_type=jnp.float32)
        e2 = jax.lax.dot_general(
            a2_ref[...], wh, (((0,), (1,)), ((), ())),
            preferred_element_type=jnp.float32)
        e2_ref[...] = e2.astype(jnp.bfloat16)
        e2s_ref[...] = (_ALPHA * e2).astype(jnp.bfloat16)
        gm = jnp.max(e2, axis=1, keepdims=True)
        z = e1t + gm
        ml = jnp.maximum(z, _ALPHA * z)
        b1_ref[...] = (e1t - ml).astype(jnp.bfloat16).T
        b2_ref[...] = (_ALPHA * e1t - ml).astype(jnp.bfloat16).T

    r0 = i * _RB
    adjf = adj_ref[...].astype(jnp.bfloat16)
    for h in range(_H):
        b1 = b1_ref[pl.ds(r0, _RB), h:h + 1]
        b2 = b2_ref[pl.ds(r0, _RB), h:h + 1]
        e2h = e2_ref[h:h + 1, :]
        e2sh = e2s_ref[h:h + 1, :]
        arg = jnp.maximum(b1 + e2h, b2 + e2sh)
        p = jnp.exp2(arg) * adjf
        whj = whext_ref[:, h * _EXT:(h + 1) * _EXT]
        res = jnp.dot(p, whj, preferred_element_type=jnp.float32)
        out_ref[:, h * _DOUT:(h + 1) * _DOUT] = (
            res[:, :_DOUT] / res[:, _DOUT:_DOUT + 1])


def kernel(x, adj, W0, a0, W1, a1, W2, a2, W3, a3):
    wcat = jnp.concatenate([W0, W1, W2, W3], axis=1)
    a1blk = jnp.zeros((_DIN, 8), jnp.float32)
    a2blk = jnp.zeros((_DIN, 8), jnp.float32)
    for h, ah in enumerate((a0, a1, a2, a3)):
        a1blk = a1blk.at[h * _DOUT:(h + 1) * _DOUT, h].set(
            _LOG2E * ah[:_DOUT, 0])
        a2blk = a2blk.at[h * _DOUT:(h + 1) * _DOUT, h].set(
            _LOG2E * ah[_DOUT:, 0])

    nrb = _N // _RB
    out = pl.pallas_call(
        _att_kernel,
        grid=(nrb,),
        in_specs=[
            pl.BlockSpec((_N, _DIN), lambda i: (0, 0)),
            pl.BlockSpec((_RB, _N), lambda i: (i, 0)),
            pl.BlockSpec((_DIN, _H * _DOUT), lambda i: (0, 0)),
            pl.BlockSpec((_DIN, 8), lambda i: (0, 0)),
            pl.BlockSpec((_DIN, 8), lambda i: (0, 0)),
        ],
        out_specs=pl.BlockSpec((_RB, _H * _DOUT), lambda i: (i, 0)),
        out_shape=jax.ShapeDtypeStruct((_N, _H * _DOUT), jnp.float32),
        scratch_shapes=[
            pltpu.VMEM((_N, _H * _EXT), jnp.bfloat16),
            pltpu.VMEM((8, _N), jnp.bfloat16),
            pltpu.VMEM((8, _N), jnp.bfloat16),
            pltpu.VMEM((_N, 8), jnp.bfloat16),
            pltpu.VMEM((_N, 8), jnp.bfloat16),
        ],
        compiler_params=pltpu.CompilerParams(
            dimension_semantics=("arbitrary",)),
    )(x, adj, wcat, a1blk, a2blk)
    return out
